# baseline (speedup 1.0000x reference)
import functools

import jax
import jax.numpy as jnp
from jax import lax
from jax.experimental import pallas as pl
from jax.experimental.pallas import tpu as pltpu
from jax.experimental.pallas import tpu_sc as plsc

_H = 16
_W = 16
_DIM = 64
_C = 8
_BN = 256


def _embed_kernel(colors_ref, seen_ref, arm_ref, ang_ref, out_ref):
    bn = colors_ref.shape[-1]
    cb = colors_ref[...]
    ct = jnp.transpose(cb, (2, 0, 1, 3))

    sb = seen_ref[...][None]

    armq = arm_ref[...] / ang_ref[...]
    armb = jnp.broadcast_to(armq[:, None, None, :], (4, _H, _W, bn))

    zeros = jnp.zeros((_DIM - 13, _H, _W, bn), jnp.float32)

    out_ref[...] = jnp.concatenate([ct, sb, armb, zeros], axis=0)


def _tc_fill(colors_p, seen_p, arm_p, ang_p, B):
    grid = (B // _BN,)
    return pl.pallas_call(
        _embed_kernel,
        grid=grid,
        in_specs=[
            pl.BlockSpec((_H, _W, _C, _BN), lambda i: (0, 0, 0, i)),
            pl.BlockSpec((_H, _W, _BN), lambda i: (0, 0, i)),
            pl.BlockSpec((4, _BN), lambda i: (0, i)),
            pl.BlockSpec((4, _BN), lambda i: (0, i)),
        ],
        out_specs=pl.BlockSpec((_DIM, _H, _W, _BN), lambda i: (0, 0, 0, i)),
        out_shape=jax.ShapeDtypeStruct((_DIM, _H, _W, B), jnp.float32),
    )(colors_p, seen_p, arm_p, ang_p)


def _make_sc_scatter(B):
    info = plsc.get_sparse_core_info()
    NC, NS, L = info.num_cores, info.num_subcores, info.num_lanes
    NW = NC * NS
    bw = B // NW
    nwords = _DIM * _H * _W * B
    mesh = plsc.VectorSubcoreMesh(core_axis_name="c", subcore_axis_name="s")

    del nwords
    @functools.partial(
        pl.kernel,
        mesh=mesh,
        out_type=(),
        scratch_types=[
            pltpu.VMEM((2, bw), jnp.int32),
            pltpu.VMEM((2, bw), jnp.int32),
            pltpu.VMEM((bw,), jnp.int32),
            pltpu.VMEM((bw,), jnp.int32),
            pltpu.VMEM((bw,), jnp.float32),
            pltpu.SemaphoreType.DMA,
        ],
    )
    def sc_scatter(loc_hbm, tgt_hbm, out_hbm, locv, tgtv, idx13, idx14, ones, sem):
        wid = lax.axis_index("s") * NC + lax.axis_index("c")
        base = wid * bw
        pltpu.sync_copy(loc_hbm.at[:, pl.ds(base, bw)], locv)
        pltpu.sync_copy(tgt_hbm.at[:, pl.ds(base, bw)], tgtv)
        for j in range(bw // L):
            sl = pl.ds(j * L, L)
            bvec = lax.iota(jnp.int32, L) + (base + j * L)
            x = locv[0, sl]
            y = locv[1, sl]
            idx13[sl] = ((13 * _H + x) * _W + y) * B + bvec
            x = tgtv[0, sl]
            y = tgtv[1, sl]
            idx14[sl] = ((14 * _H + x) * _W + y) * B + bvec
            ones[sl] = jnp.ones((L,), jnp.float32)
        pltpu.async_copy(ones, out_hbm.at[idx13], sem).wait()
        pltpu.async_copy(ones, out_hbm.at[idx14], sem).wait()

    return sc_scatter


@jax.jit
def kernel(colors, seen, arm, angle_sizes, loc, target):
    B = colors.shape[0]
    colors_p = jnp.transpose(colors, (1, 2, 3, 0))
    seen_p = jnp.transpose(seen, (1, 2, 0))
    arm_p = jnp.transpose(arm, (1, 0))
    loc_p = jnp.transpose(loc, (1, 0))
    tgt_p = jnp.transpose(target, (1, 0))
    ang_p = jnp.broadcast_to(angle_sizes[:, None], (4, B))

    out = _tc_fill(colors_p, seen_p, arm_p, ang_p, B)
    out_ref = jax.new_ref(out.reshape(-1))
    _make_sc_scatter(B)(loc_p, tgt_p, out_ref)
    out = out_ref[...].reshape(_DIM, _H, _W, B)
    return jnp.transpose(out, (3, 0, 1, 2))


# speedup vs baseline: 3.4015x; 3.4015x over previous
import functools

import jax
import jax.numpy as jnp
from jax import lax
from jax.experimental import pallas as pl
from jax.experimental.pallas import tpu as pltpu
from jax.experimental.pallas import tpu_sc as plsc

_H = 16
_W = 16
_DIM = 64
_C = 8
_BN = 256


def _embed_kernel(colors_ref, seen_ref, arm_ref, ang_ref, oh_ref, out_ref):
    bn = colors_ref.shape[-1]
    cb = colors_ref[...]
    ct = jnp.transpose(cb, (2, 0, 1, 3))

    sb = seen_ref[...][None]

    armq = arm_ref[...] / ang_ref[...]
    armb = jnp.broadcast_to(armq[:, None, None, :], (4, _H, _W, bn))

    oh = oh_ref[...]

    zeros = jnp.zeros((_DIM - 15, _H, _W, bn), jnp.float32)

    out_ref[...] = jnp.concatenate([ct, sb, armb, oh, zeros], axis=0)


def _tc_fill(colors_p, seen_p, arm_p, ang_p, oh_p, B):
    grid = (B // _BN,)
    return pl.pallas_call(
        _embed_kernel,
        grid=grid,
        in_specs=[
            pl.BlockSpec((_H, _W, _C, _BN), lambda i: (0, 0, 0, i)),
            pl.BlockSpec((_H, _W, _BN), lambda i: (0, 0, i)),
            pl.BlockSpec((4, _BN), lambda i: (0, i)),
            pl.BlockSpec((4, _BN), lambda i: (0, i)),
            pl.BlockSpec((2, _H, _W, _BN), lambda i: (0, 0, 0, i)),
        ],
        out_specs=pl.BlockSpec((_DIM, _H, _W, _BN), lambda i: (0, 0, 0, i)),
        out_shape=jax.ShapeDtypeStruct((_DIM, _H, _W, B), jnp.float32),
    )(colors_p, seen_p, arm_p, ang_p, oh_p)


def _make_sc_onehot(B):
    info = plsc.get_sparse_core_info()
    NC, NS, L = info.num_cores, info.num_subcores, info.num_lanes
    NW = NC * NS
    bw = B // NW
    slab_words = 2 * _H * _W * bw
    mesh = plsc.VectorSubcoreMesh(core_axis_name="c", subcore_axis_name="s")

    @functools.partial(
        pl.kernel,
        mesh=mesh,
        compiler_params=pltpu.CompilerParams(needs_layout_passes=False),
        out_type=jax.ShapeDtypeStruct((NW, slab_words), jnp.float32),
        scratch_types=[
            pltpu.VMEM((2, bw), jnp.int32),
            pltpu.VMEM((2, bw), jnp.int32),
            pltpu.VMEM((slab_words,), jnp.float32),
        ],
    )
    def sc_onehot(loc_hbm, tgt_hbm, out_hbm, locv, tgtv, slab):
        wid = lax.axis_index("s") * NC + lax.axis_index("c")
        base = wid * bw
        pltpu.sync_copy(loc_hbm.at[:, pl.ds(base, bw)], locv)
        pltpu.sync_copy(tgt_hbm.at[:, pl.ds(base, bw)], tgtv)

        @pl.loop(0, slab_words // L)
        def _zero(j):
            slab[pl.ds(j * L, L)] = jnp.zeros((L,), jnp.float32)

        ones = jnp.ones((L,), jnp.float32)
        half = _H * _W * bw
        for j in range(bw // L):
            sl = pl.ds(j * L, L)
            bvec = lax.iota(jnp.int32, L) + j * L
            idx13 = (locv[0, sl] * _W + locv[1, sl]) * bw + bvec
            idx14 = (tgtv[0, sl] * _W + tgtv[1, sl]) * bw + bvec + half
            plsc.store_scatter(slab, [idx13], ones)
            plsc.store_scatter(slab, [idx14], ones)
        pltpu.sync_copy(slab, out_hbm.at[wid])

    return sc_onehot, NW, bw


@jax.jit
def kernel(colors, seen, arm, angle_sizes, loc, target):
    B = colors.shape[0]
    colors_p = jnp.transpose(colors, (1, 2, 3, 0))
    seen_p = jnp.transpose(seen, (1, 2, 0))
    arm_p = jnp.transpose(arm, (1, 0))
    loc_p = jnp.transpose(loc, (1, 0))
    tgt_p = jnp.transpose(target, (1, 0))
    ang_p = jnp.broadcast_to(angle_sizes[:, None], (4, B))

    sc_onehot, NW, bw = _make_sc_onehot(B)
    oh_w = sc_onehot(loc_p, tgt_p).reshape(NW, 2, _H, _W, bw)
    oh_p = jnp.transpose(oh_w, (1, 2, 3, 0, 4)).reshape(2, _H, _W, B)

    out = _tc_fill(colors_p, seen_p, arm_p, ang_p, oh_p, B)
    return jnp.transpose(out, (3, 0, 1, 2))


# speedup vs baseline: 6.0803x; 1.7875x over previous
import jax
import jax.numpy as jnp
from jax.experimental import pallas as pl

_H = 16
_W = 16
_DIM = 64
_C = 8
_BN = 256


def _embed_kernel(colors_ref, seen_ref, arm_ref, ang_ref, loc_ref, tgt_ref, out_ref):
    bn = colors_ref.shape[-1]
    cb = colors_ref[...]
    ct = jnp.transpose(cb, (2, 0, 1, 3))

    sb = seen_ref[...][None]

    armq = arm_ref[...] / ang_ref[...]
    armb = jnp.broadcast_to(armq[:, None, None, :], (4, _H, _W, bn))

    h_i = jax.lax.broadcasted_iota(jnp.int32, (1, _H, _W, bn), 1)
    w_i = jax.lax.broadcasted_iota(jnp.int32, (1, _H, _W, bn), 2)
    loc = loc_ref[...]
    l1h = ((h_i == loc[0][None, None, None, :])
           & (w_i == loc[1][None, None, None, :])).astype(jnp.float32)
    tgt = tgt_ref[...]
    t1h = ((h_i == tgt[0][None, None, None, :])
           & (w_i == tgt[1][None, None, None, :])).astype(jnp.float32)

    zeros = jnp.zeros((_DIM - 15, _H, _W, bn), jnp.float32)

    out_ref[...] = jnp.concatenate([ct, sb, armb, l1h, t1h, zeros], axis=0)


@jax.jit
def kernel(colors, seen, arm, angle_sizes, loc, target):
    B = colors.shape[0]
    colors_p = jnp.transpose(colors, (1, 2, 3, 0))
    seen_p = jnp.transpose(seen, (1, 2, 0))
    arm_p = jnp.transpose(arm, (1, 0))
    loc_p = jnp.transpose(loc, (1, 0))
    tgt_p = jnp.transpose(target, (1, 0))
    ang_p = jnp.broadcast_to(angle_sizes[:, None], (4, B))

    grid = (B // _BN,)
    out = pl.pallas_call(
        _embed_kernel,
        grid=grid,
        in_specs=[
            pl.BlockSpec((_H, _W, _C, _BN), lambda i: (0, 0, 0, i)),
            pl.BlockSpec((_H, _W, _BN), lambda i: (0, 0, i)),
            pl.BlockSpec((4, _BN), lambda i: (0, i)),
            pl.BlockSpec((4, _BN), lambda i: (0, i)),
            pl.BlockSpec((2, _BN), lambda i: (0, i)),
            pl.BlockSpec((2, _BN), lambda i: (0, i)),
        ],
        out_specs=pl.BlockSpec((_DIM, _H, _W, _BN), lambda i: (0, 0, 0, i)),
        out_shape=jax.ShapeDtypeStruct((_DIM, _H, _W, B), jnp.float32),
    )(colors_p, seen_p, arm_p, ang_p, loc_p, tgt_p)
    return jnp.transpose(out, (3, 0, 1, 2))
